# baseline (speedup 1.0000x reference)
import functools

import jax
import jax.numpy as jnp
from jax.experimental import pallas as pl
from jax.experimental.pallas import tpu as pltpu

_BN = 2048
_SLOTS = 3


def _stats_body(t_ref, q_ref, w_ref, train_ref, m_ref, s_ref, zq_ref, *,
                n_total):
    nb = pl.program_id(0)

    @pl.when(nb == 0)
    def _init():
        zq_ref[...] = jnp.dot(q_ref[...], w_ref[...],
                              preferred_element_type=jnp.float32)
        m_ref[...] = jnp.full_like(m_ref, -1e30)
        s_ref[...] = jnp.zeros_like(s_ref)

    zt = jnp.dot(train_ref[...], w_ref[...],
                 preferred_element_type=jnp.float32)
    sim = jnp.dot(zq_ref[...], zt.T, preferred_element_type=jnp.float32)
    logits = sim / t_ref[0]
    col = nb * _BN + jax.lax.broadcasted_iota(jnp.int32, logits.shape, 1)
    logits = jnp.where(col < n_total, logits, -1e30)

    m_old = m_ref[...]
    m_new = jnp.maximum(m_old, jnp.max(logits, axis=1, keepdims=True))
    s_ref[...] = (s_ref[...] * jnp.exp(m_old - m_new)
                  + jnp.sum(jnp.exp(logits - m_new), axis=1, keepdims=True))
    m_ref[...] = m_new


def _combine_body(t_ref, q_ref, w_ref, train_ref, act_ref, m_ref, s_ref,
                  wout_ref, tail_ref, pred_ref, zq_ref, wbuf_ref, sem, *,
                  n_total, nb_total):
    nb = pl.program_id(0)

    @pl.when(nb == 0)
    def _init():
        zq_ref[...] = jnp.dot(q_ref[...], w_ref[...],
                              preferred_element_type=jnp.float32)
        pred_ref[...] = jnp.zeros_like(pred_ref)

    slot = jax.lax.rem(nb, _SLOTS)

    @pl.when(jnp.logical_and(nb >= _SLOTS, nb - _SLOTS <= nb_total - 2))
    def _wait_prev():
        pltpu.make_async_copy(
            wbuf_ref.at[slot],
            wout_ref.at[:, pl.ds((nb - _SLOTS) * _BN, _BN)],
            sem.at[slot],
        ).wait()

    zt = jnp.dot(train_ref[...], w_ref[...],
                 preferred_element_type=jnp.float32)
    sim = jnp.dot(zq_ref[...], zt.T, preferred_element_type=jnp.float32)
    logits = sim / t_ref[0]
    col = nb * _BN + jax.lax.broadcasted_iota(jnp.int32, logits.shape, 1)
    w = jnp.exp(logits - m_ref[...]) / s_ref[...]
    w = jnp.where(col < n_total, w, 0.0)

    @pl.when(nb < nb_total - 1)
    def _start_full():
        wbuf_ref[slot] = w
        pltpu.make_async_copy(
            wbuf_ref.at[slot],
            wout_ref.at[:, pl.ds(nb * _BN, _BN)],
            sem.at[slot],
        ).start()

    @pl.when(nb == nb_total - 1)
    def _emit_tail():
        tail_ref[...] = w

    arow = jax.lax.broadcasted_iota(jnp.int32, act_ref.shape, 0) + nb * _BN
    act = jnp.where(arow < n_total, act_ref[...], 0.0)
    pred_ref[...] += jnp.dot(w, act, preferred_element_type=jnp.float32)

    @pl.when(nb == nb_total - 1)
    def _drain():
        for j in range(max(0, nb_total - _SLOTS), nb_total - 1):
            s_slot = j % _SLOTS
            pltpu.make_async_copy(
                wbuf_ref.at[s_slot],
                wout_ref.at[:, pl.ds(j * _BN, _BN)],
                sem.at[s_slot],
            ).wait()


def _tail_patch_body(wmain_ref, tail_ref, wout_ref):
    wout_ref[...] = tail_ref[...]


def kernel(query_obs, train_obs, train_actions, W_enc, log_temperature):
    B, d = query_obs.shape
    N = train_obs.shape[0]
    H, A = train_actions.shape[1], train_actions.shape[2]
    HA = H * A
    nb_total = pl.cdiv(N, _BN)

    temp = jnp.exp(log_temperature).reshape(1)
    act_flat = train_actions.reshape(N, HA)

    scalar_spec = pl.BlockSpec(memory_space=pltpu.SMEM)

    m, s = pl.pallas_call(
        functools.partial(_stats_body, n_total=N),
        grid=(nb_total,),
        in_specs=[
            scalar_spec,
            pl.BlockSpec((B, d), lambda nb: (0, 0)),
            pl.BlockSpec((d, d), lambda nb: (0, 0)),
            pl.BlockSpec((_BN, d), lambda nb: (nb, 0)),
        ],
        out_specs=[
            pl.BlockSpec((B, 1), lambda nb: (0, 0)),
            pl.BlockSpec((B, 1), lambda nb: (0, 0)),
        ],
        out_shape=[
            jax.ShapeDtypeStruct((B, 1), jnp.float32),
            jax.ShapeDtypeStruct((B, 1), jnp.float32),
        ],
        scratch_shapes=[pltpu.VMEM((B, d), jnp.float32)],
        compiler_params=pltpu.CompilerParams(
            dimension_semantics=("arbitrary",),
        ),
    )(temp, query_obs, W_enc, train_obs)

    w_main, w_tail, pred = pl.pallas_call(
        functools.partial(_combine_body, n_total=N, nb_total=nb_total),
        grid=(nb_total,),
        in_specs=[
            scalar_spec,
            pl.BlockSpec((B, d), lambda nb: (0, 0)),
            pl.BlockSpec((d, d), lambda nb: (0, 0)),
            pl.BlockSpec((_BN, d), lambda nb: (nb, 0)),
            pl.BlockSpec((_BN, HA), lambda nb: (nb, 0)),
            pl.BlockSpec((B, 1), lambda nb: (0, 0)),
            pl.BlockSpec((B, 1), lambda nb: (0, 0)),
        ],
        out_specs=[
            pl.BlockSpec(memory_space=pltpu.MemorySpace.HBM),
            pl.BlockSpec((B, _BN), lambda nb: (0, 0)),
            pl.BlockSpec((B, HA), lambda nb: (0, 0)),
        ],
        out_shape=[
            jax.ShapeDtypeStruct((B, N), jnp.float32),
            jax.ShapeDtypeStruct((B, _BN), jnp.float32),
            jax.ShapeDtypeStruct((B, HA), jnp.float32),
        ],
        scratch_shapes=[
            pltpu.VMEM((B, d), jnp.float32),
            pltpu.VMEM((_SLOTS, B, _BN), jnp.float32),
            pltpu.SemaphoreType.DMA((_SLOTS,)),
        ],
        compiler_params=pltpu.CompilerParams(
            dimension_semantics=("arbitrary",),
        ),
    )(temp, query_obs, W_enc, train_obs, act_flat, m, s)

    tail_block = nb_total - 1
    weights = pl.pallas_call(
        _tail_patch_body,
        grid=(1,),
        in_specs=[
            pl.BlockSpec(memory_space=pltpu.MemorySpace.HBM),
            pl.BlockSpec((B, _BN), lambda i: (0, 0)),
        ],
        out_specs=pl.BlockSpec((B, _BN), lambda i: (0, tail_block)),
        out_shape=jax.ShapeDtypeStruct((B, N), jnp.float32),
        input_output_aliases={0: 0},
    )(w_main, w_tail)

    return (pred.reshape(B, H, A), weights)


# speedup vs baseline: 1.0040x; 1.0040x over previous
import functools

import jax
import jax.numpy as jnp
from jax.experimental import pallas as pl
from jax.experimental.pallas import tpu as pltpu

_BN = 2048
_SLOTS = 4


def _stats_body(t_ref, q_ref, w_ref, train_ref, m_ref, s_ref, zq_ref, *,
                n_total):
    nb = pl.program_id(0)

    @pl.when(nb == 0)
    def _init():
        zq_ref[...] = jnp.dot(q_ref[...], w_ref[...],
                              preferred_element_type=jnp.float32)
        m_ref[...] = jnp.full_like(m_ref, -1e30)
        s_ref[...] = jnp.zeros_like(s_ref)

    zt = jnp.dot(train_ref[...], w_ref[...],
                 preferred_element_type=jnp.float32)
    sim = jnp.dot(zq_ref[...], zt.T, preferred_element_type=jnp.float32)
    logits = sim / t_ref[0]
    col = nb * _BN + jax.lax.broadcasted_iota(jnp.int32, logits.shape, 1)
    logits = jnp.where(col < n_total, logits, -1e30)

    m_old = m_ref[...]
    m_new = jnp.maximum(m_old, jnp.max(logits, axis=1, keepdims=True))
    s_ref[...] = (s_ref[...] * jnp.exp(m_old - m_new)
                  + jnp.sum(jnp.exp(logits - m_new), axis=1, keepdims=True))
    m_ref[...] = m_new


def _combine_body(t_ref, q_ref, w_ref, train_ref, act_ref, m_ref, s_ref,
                  wout_ref, pred_ref, zq_ref, wbuf_ref, sem, *,
                  n_total, nb_total):
    nb = pl.program_id(0)

    @pl.when(nb == 0)
    def _init():
        zq_ref[...] = jnp.dot(q_ref[...], w_ref[...],
                              preferred_element_type=jnp.float32)
        pred_ref[...] = jnp.zeros_like(pred_ref)

    slot = jax.lax.rem(nb, _SLOTS)

    @pl.when(jnp.logical_and(nb >= _SLOTS, nb - _SLOTS <= nb_total - 2))
    def _wait_prev():
        pltpu.make_async_copy(
            wbuf_ref.at[slot],
            wout_ref.at[:, pl.ds((nb - _SLOTS) * _BN, _BN)],
            sem.at[slot],
        ).wait()

    zt = jnp.dot(train_ref[...], w_ref[...],
                 preferred_element_type=jnp.float32)
    sim = jnp.dot(zq_ref[...], zt.T, preferred_element_type=jnp.float32)
    logits = sim / t_ref[0]
    col = nb * _BN + jax.lax.broadcasted_iota(jnp.int32, logits.shape, 1)
    w = jnp.exp(logits - m_ref[...]) / s_ref[...]
    w = jnp.where(col < n_total, w, 0.0)

    @pl.when(nb < nb_total - 1)
    def _start_full():
        wbuf_ref[slot] = w
        pltpu.make_async_copy(
            wbuf_ref.at[slot],
            wout_ref.at[:, pl.ds(nb * _BN, _BN)],
            sem.at[slot],
        ).start()

    arow = jax.lax.broadcasted_iota(jnp.int32, act_ref.shape, 0) + nb * _BN
    act = jnp.where(arow < n_total, act_ref[...], 0.0)
    pred_ref[...] += jnp.dot(w, act, preferred_element_type=jnp.float32)

    @pl.when(nb == nb_total - 1)
    def _drain():
        for j in range(max(0, nb_total - _SLOTS), nb_total - 1):
            s_slot = j % _SLOTS
            pltpu.make_async_copy(
                wbuf_ref.at[s_slot],
                wout_ref.at[:, pl.ds(j * _BN, _BN)],
                sem.at[s_slot],
            ).wait()


def _tail_patch_body(t_ref, wmain_ref, q_ref, w_ref, train_ref, m_ref, s_ref,
                     wout_ref, *, n_total, nb_total):
    zq = jnp.dot(q_ref[...], w_ref[...], preferred_element_type=jnp.float32)
    zt = jnp.dot(train_ref[...], w_ref[...],
                 preferred_element_type=jnp.float32)
    sim = jnp.dot(zq, zt.T, preferred_element_type=jnp.float32)
    logits = sim / t_ref[0]
    col = ((nb_total - 1) * _BN
           + jax.lax.broadcasted_iota(jnp.int32, logits.shape, 1))
    w = jnp.exp(logits - m_ref[...]) / s_ref[...]
    wout_ref[...] = jnp.where(col < n_total, w, 0.0)


def kernel(query_obs, train_obs, train_actions, W_enc, log_temperature):
    B, d = query_obs.shape
    N = train_obs.shape[0]
    H, A = train_actions.shape[1], train_actions.shape[2]
    HA = H * A
    nb_total = pl.cdiv(N, _BN)

    temp = jnp.exp(log_temperature).reshape(1)
    act_flat = train_actions.reshape(N, HA)

    scalar_spec = pl.BlockSpec(memory_space=pltpu.SMEM)

    m, s = pl.pallas_call(
        functools.partial(_stats_body, n_total=N),
        grid=(nb_total,),
        in_specs=[
            scalar_spec,
            pl.BlockSpec((B, d), lambda nb: (0, 0)),
            pl.BlockSpec((d, d), lambda nb: (0, 0)),
            pl.BlockSpec((_BN, d), lambda nb: (nb, 0)),
        ],
        out_specs=[
            pl.BlockSpec((B, 1), lambda nb: (0, 0)),
            pl.BlockSpec((B, 1), lambda nb: (0, 0)),
        ],
        out_shape=[
            jax.ShapeDtypeStruct((B, 1), jnp.float32),
            jax.ShapeDtypeStruct((B, 1), jnp.float32),
        ],
        scratch_shapes=[pltpu.VMEM((B, d), jnp.float32)],
        compiler_params=pltpu.CompilerParams(
            dimension_semantics=("arbitrary",),
        ),
    )(temp, query_obs, W_enc, train_obs)

    w_main, pred = pl.pallas_call(
        functools.partial(_combine_body, n_total=N, nb_total=nb_total),
        grid=(nb_total,),
        in_specs=[
            scalar_spec,
            pl.BlockSpec((B, d), lambda nb: (0, 0)),
            pl.BlockSpec((d, d), lambda nb: (0, 0)),
            pl.BlockSpec((_BN, d), lambda nb: (nb, 0)),
            pl.BlockSpec((_BN, HA), lambda nb: (nb, 0)),
            pl.BlockSpec((B, 1), lambda nb: (0, 0)),
            pl.BlockSpec((B, 1), lambda nb: (0, 0)),
        ],
        out_specs=[
            pl.BlockSpec(memory_space=pltpu.MemorySpace.HBM),
            pl.BlockSpec((B, HA), lambda nb: (0, 0)),
        ],
        out_shape=[
            jax.ShapeDtypeStruct((B, N), jnp.float32),
            jax.ShapeDtypeStruct((B, HA), jnp.float32),
        ],
        scratch_shapes=[
            pltpu.VMEM((B, d), jnp.float32),
            pltpu.VMEM((_SLOTS, B, _BN), jnp.float32),
            pltpu.SemaphoreType.DMA((_SLOTS,)),
        ],
        compiler_params=pltpu.CompilerParams(
            dimension_semantics=("arbitrary",),
        ),
    )(temp, query_obs, W_enc, train_obs, act_flat, m, s)

    tail_block = nb_total - 1
    weights = pl.pallas_call(
        functools.partial(_tail_patch_body, n_total=N, nb_total=nb_total),
        grid=(1,),
        in_specs=[
            scalar_spec,
            pl.BlockSpec(memory_space=pltpu.MemorySpace.HBM),
            pl.BlockSpec((B, d), lambda i: (0, 0)),
            pl.BlockSpec((d, d), lambda i: (0, 0)),
            pl.BlockSpec((_BN, d), lambda i: (tail_block, 0)),
            pl.BlockSpec((B, 1), lambda i: (0, 0)),
            pl.BlockSpec((B, 1), lambda i: (0, 0)),
        ],
        out_specs=pl.BlockSpec((B, _BN), lambda i: (0, tail_block)),
        out_shape=jax.ShapeDtypeStruct((B, N), jnp.float32),
        input_output_aliases={1: 0},
    )(temp, w_main, query_obs, W_enc, train_obs, m, s)

    return (pred.reshape(B, H, A), weights)
